# baseline (speedup 1.0000x reference)
import functools

import jax
import jax.numpy as jnp
from jax import lax
from jax.experimental import pallas as pl
from jax.experimental.pallas import tpu as pltpu
from jax.experimental.pallas import tpu_sc as plsc

N, E, D = 10000, 320000, 128
NC, NS = 2, 16
NW = NC * NS
EPW = E // NW
B = 128
NBP = 80
PADE = NBP * B - EPW
K = 2
NSTEP = NBP // K
NPAD = 10240
RPT = NPAD // NS


@functools.partial(
    pl.kernel,
    out_type=jax.ShapeDtypeStruct((NC, NPAD, D), jnp.float32),
    mesh=plsc.VectorSubcoreMesh(core_axis_name="c", subcore_axis_name="s"),
    scratch_types=(
        [pltpu.VMEM((K, B), jnp.int32)] * 2
        + [pltpu.VMEM((K, B), jnp.int32)] * 2
        + [pltpu.VMEM((B, D), jnp.float32)] * K
        + [pltpu.VMEM_SHARED((NPAD, D), jnp.float32)]
        + [pltpu.SemaphoreType.DMA] * (K + 2)
    ),
)
def _sc_agg(x_hbm, src_hbm, dst_hbm, out_hbm, *rest):
    sidx = rest[0:2]
    didx = rest[2:4]
    rows = rest[4:4 + K]
    acc_sh = rest[4 + K]
    gsem = rest[5 + K:5 + 2 * K]
    isem = rest[5 + 2 * K:7 + 2 * K]

    c = lax.axis_index("c")
    s = lax.axis_index("s")
    wid = s * NC + c

    def fetch_idx(i, p):
        pltpu.async_copy(src_hbm.at[wid, i], sidx[p], isem[p])
        pltpu.async_copy(dst_hbm.at[wid, i], didx[p], isem[p])

    def drain_idx(p):
        pltpu.make_async_copy(src_hbm.at[wid, 0], sidx[p], isem[p]).wait()
        pltpu.make_async_copy(dst_hbm.at[wid, 0], didx[p], isem[p]).wait()

    fetch_idx(0, 0)

    zeros16 = jnp.zeros((16,), jnp.float32)

    def zero_buf(j, carry):
        rows[0][j // (D // 16), pl.ds((j % (D // 16)) * 16, 16)] = zeros16
        return carry

    lax.fori_loop(0, B * (D // 16), zero_buf, 0)
    r0 = s * RPT

    def zero_acc(i, carry):
        pltpu.sync_copy(rows[0], acc_sh.at[pl.ds(r0 + i * B, B)])
        return carry

    lax.fori_loop(0, RPT // B, zero_acc, 0)
    plsc.subcore_barrier()

    def step(i, carry):
        p = lax.rem(i, 2)

        @pl.when(i + 1 < NSTEP)
        def _prefetch():
            @pl.when(p == 0)
            def _():
                fetch_idx(i + 1, 1)

            @pl.when(p == 1)
            def _():
                fetch_idx(i + 1, 0)

        def run_set(p):
            drain_idx(p)
            gd = [
                pltpu.async_copy(x_hbm.at[sidx[p].at[j]], rows[j], gsem[j])
                for j in range(K)
            ]
            for j in range(K):
                gd[j].wait()
                pltpu.sync_copy(rows[j], acc_sh.at[didx[p].at[j]], add=True)

        @pl.when(p == 0)
        def _():
            run_set(0)

        @pl.when(p == 1)
        def _():
            run_set(1)

        return carry

    lax.fori_loop(0, NSTEP, step, 0)
    plsc.subcore_barrier()

    pltpu.sync_copy(acc_sh.at[pl.ds(r0, RPT)], out_hbm.at[c, pl.ds(r0, RPT)])


def _mlp_body(x_ref, p_ref, w1_ref, b1_ref, w2_ref, b2_ref, o_ref):
    h = x_ref[...] + p_ref[0] + p_ref[1]
    h1 = jnp.dot(h, w1_ref[...], preferred_element_type=jnp.float32)
    h1 = jnp.maximum(h1 + b1_ref[...], 0.0)
    o_ref[...] = (
        jnp.dot(h1, w2_ref[...], preferred_element_type=jnp.float32)
        + b2_ref[...]
    )


_ROWS_BLK = 1000


def _mlp(x, partials, W1, b1, W2, b2):
    grid = (N // _ROWS_BLK,)
    return pl.pallas_call(
        _mlp_body,
        grid=grid,
        in_specs=[
            pl.BlockSpec((_ROWS_BLK, D), lambda i: (i, 0)),
            pl.BlockSpec((NC, _ROWS_BLK, D), lambda i: (0, i, 0)),
            pl.BlockSpec((D, D), lambda i: (0, 0)),
            pl.BlockSpec((1, D), lambda i: (0, 0)),
            pl.BlockSpec((D, D), lambda i: (0, 0)),
            pl.BlockSpec((1, D), lambda i: (0, 0)),
        ],
        out_specs=pl.BlockSpec((_ROWS_BLK, D), lambda i: (i, 0)),
        out_shape=jax.ShapeDtypeStruct((N, D), jnp.float32),
    )(x, partials, W1, b1, W2, b2)


def kernel(x, edge_index, W1, b1, W2, b2):
    src = edge_index[0].reshape(NW, EPW)
    dst = edge_index[1].reshape(NW, EPW)
    src = jnp.concatenate(
        [src, jnp.zeros((NW, PADE), jnp.int32)], axis=1
    ).reshape(NW, NSTEP, K, B)
    pad_dst = jnp.broadcast_to(
        N + (jnp.arange(PADE, dtype=jnp.int32) % (NPAD - N)), (NW, PADE))
    dst = jnp.concatenate([dst, pad_dst], axis=1).reshape(NW, NSTEP, K, B)
    partials = _sc_agg(x, src, dst)
    return _mlp(x, partials, W1, b1.reshape(1, D), W2, b2.reshape(1, D))


# speedup vs baseline: 1.6633x; 1.6633x over previous
import functools

import jax
import jax.numpy as jnp
from jax import lax
from jax.experimental import pallas as pl
from jax.experimental.pallas import tpu as pltpu
from jax.experimental.pallas import tpu_sc as plsc

N, E, D = 10000, 320000, 128
NC, NS = 2, 16
NW = NC * NS
EPW = E // NW
B = 112
NBP = 90
PADE = NBP * B - EPW
K = 2
NSTEP = NBP // K
NPAD = 10240
RPT = NPAD // NS


@functools.partial(
    pl.kernel,
    out_type=jax.ShapeDtypeStruct((NC, NPAD, D), jnp.float32),
    mesh=plsc.VectorSubcoreMesh(core_axis_name="c", subcore_axis_name="s"),
    scratch_types=(
        [pltpu.VMEM((K, B), jnp.int32)] * 2
        + [pltpu.VMEM((K, B), jnp.int32)] * 2
        + [pltpu.VMEM((B, D), jnp.float32)] * K
        + [pltpu.VMEM_SHARED((NPAD, D), jnp.float32)]
        + [pltpu.SemaphoreType.DMA] * (K + 2)
    ),
)
def _sc_agg(x_hbm, src_hbm, dst_hbm, out_hbm, *rest):
    sidx = rest[0:2]
    didx = rest[2:4]
    rows = rest[4:4 + K]
    acc_sh = rest[4 + K]
    gsem = rest[5 + K:5 + 2 * K]
    isem = rest[5 + 2 * K:7 + 2 * K]

    c = lax.axis_index("c")
    s = lax.axis_index("s")
    wid = s * NC + c

    def fetch_idx(i, p):
        pltpu.async_copy(src_hbm.at[wid, i], sidx[p], isem[p])
        pltpu.async_copy(dst_hbm.at[wid, i], didx[p], isem[p])

    def drain_idx(p):
        pltpu.make_async_copy(src_hbm.at[wid, 0], sidx[p], isem[p]).wait()
        pltpu.make_async_copy(dst_hbm.at[wid, 0], didx[p], isem[p]).wait()

    fetch_idx(0, 0)

    zeros16 = jnp.zeros((16,), jnp.float32)

    def zero_buf(j, carry):
        rows[0][j // (D // 16), pl.ds((j % (D // 16)) * 16, 16)] = zeros16
        return carry

    lax.fori_loop(0, B * (D // 16), zero_buf, 0)
    r0 = s * RPT

    def zero_acc(i, carry):
        pltpu.sync_copy(rows[0], acc_sh.at[pl.ds(r0 + i * B, B)])
        return carry

    nfull = RPT // B
    lax.fori_loop(0, nfull, zero_acc, 0)
    tail = RPT - nfull * B
    pltpu.sync_copy(rows[0].at[pl.ds(0, tail)],
                    acc_sh.at[pl.ds(r0 + nfull * B, tail)])
    plsc.subcore_barrier()

    def step(i, carry):
        p = lax.rem(i, 2)

        @pl.when(i + 1 < NSTEP)
        def _prefetch():
            @pl.when(p == 0)
            def _():
                fetch_idx(i + 1, 1)

            @pl.when(p == 1)
            def _():
                fetch_idx(i + 1, 0)

        def run_set(p):
            drain_idx(p)
            gd = [
                pltpu.async_copy(x_hbm.at[sidx[p].at[j]], rows[j], gsem[j])
                for j in range(K)
            ]
            for j in range(K):
                gd[j].wait()
                pltpu.sync_copy(rows[j], acc_sh.at[didx[p].at[j]], add=True)

        @pl.when(p == 0)
        def _():
            run_set(0)

        @pl.when(p == 1)
        def _():
            run_set(1)

        return carry

    lax.fori_loop(0, NSTEP, step, 0)
    plsc.subcore_barrier()

    pltpu.sync_copy(acc_sh.at[pl.ds(r0, RPT)], out_hbm.at[c, pl.ds(r0, RPT)])


def _mlp_body(x_ref, p_ref, w1_ref, b1_ref, w2_ref, b2_ref, o_ref):
    h = x_ref[...] + p_ref[0] + p_ref[1]
    h1 = jnp.dot(h, w1_ref[...], preferred_element_type=jnp.float32)
    h1 = jnp.maximum(h1 + b1_ref[...], 0.0)
    o_ref[...] = (
        jnp.dot(h1, w2_ref[...], preferred_element_type=jnp.float32)
        + b2_ref[...]
    )


_ROWS_BLK = 1000


def _mlp(x, partials, W1, b1, W2, b2):
    grid = (N // _ROWS_BLK,)
    return pl.pallas_call(
        _mlp_body,
        grid=grid,
        in_specs=[
            pl.BlockSpec((_ROWS_BLK, D), lambda i: (i, 0)),
            pl.BlockSpec((NC, _ROWS_BLK, D), lambda i: (0, i, 0)),
            pl.BlockSpec((D, D), lambda i: (0, 0)),
            pl.BlockSpec((1, D), lambda i: (0, 0)),
            pl.BlockSpec((D, D), lambda i: (0, 0)),
            pl.BlockSpec((1, D), lambda i: (0, 0)),
        ],
        out_specs=pl.BlockSpec((_ROWS_BLK, D), lambda i: (i, 0)),
        out_shape=jax.ShapeDtypeStruct((N, D), jnp.float32),
    )(x, partials, W1, b1, W2, b2)


def kernel(x, edge_index, W1, b1, W2, b2):
    src = edge_index[0].reshape(NW, EPW)
    dst = edge_index[1].reshape(NW, EPW)
    src = jnp.concatenate(
        [src, jnp.zeros((NW, PADE), jnp.int32)], axis=1
    ).reshape(NW, NSTEP, K, B)
    pad_dst = jnp.broadcast_to(
        N + (jnp.arange(PADE, dtype=jnp.int32) % (NPAD - N)), (NW, PADE))
    dst = jnp.concatenate([dst, pad_dst], axis=1).reshape(NW, NSTEP, K, B)
    partials = _sc_agg(x, src, dst)
    return _mlp(x, partials, W1, b1.reshape(1, D), W2, b2.reshape(1, D))


# speedup vs baseline: 1.6980x; 1.0208x over previous
import functools

import jax
import jax.numpy as jnp
from jax import lax
from jax.experimental import pallas as pl
from jax.experimental.pallas import tpu as pltpu
from jax.experimental.pallas import tpu_sc as plsc

N, E, D = 10000, 320000, 128
NC, NS = 2, 16
NW = NC * NS
EPW = E // NW
B = 120
NBP = 84
PADE = NBP * B - EPW
K = 3
NSTEP = NBP // K
NPAD = 10240
RPT = NPAD // NS


@functools.partial(
    pl.kernel,
    out_type=jax.ShapeDtypeStruct((NC, NPAD, D), jnp.float32),
    mesh=plsc.VectorSubcoreMesh(core_axis_name="c", subcore_axis_name="s"),
    scratch_types=(
        [pltpu.VMEM((K, B), jnp.int32)] * 2
        + [pltpu.VMEM((K, B), jnp.int32)] * 2
        + [pltpu.VMEM((B, D), jnp.float32)] * K
        + [pltpu.VMEM_SHARED((NPAD, D), jnp.float32)]
        + [pltpu.SemaphoreType.DMA] * (K + 2)
    ),
)
def _sc_agg(x_hbm, src_hbm, dst_hbm, out_hbm, *rest):
    sidx = rest[0:2]
    didx = rest[2:4]
    rows = rest[4:4 + K]
    acc_sh = rest[4 + K]
    gsem = rest[5 + K:5 + 2 * K]
    isem = rest[5 + 2 * K:7 + 2 * K]

    c = lax.axis_index("c")
    s = lax.axis_index("s")
    wid = s * NC + c

    def fetch_idx(i, p):
        pltpu.async_copy(src_hbm.at[wid, i], sidx[p], isem[p])
        pltpu.async_copy(dst_hbm.at[wid, i], didx[p], isem[p])

    def drain_idx(p):
        pltpu.make_async_copy(src_hbm.at[wid, 0], sidx[p], isem[p]).wait()
        pltpu.make_async_copy(dst_hbm.at[wid, 0], didx[p], isem[p]).wait()

    fetch_idx(0, 0)

    zeros16 = jnp.zeros((16,), jnp.float32)

    def zero_buf(j, carry):
        rows[0][j // (D // 16), pl.ds((j % (D // 16)) * 16, 16)] = zeros16
        return carry

    lax.fori_loop(0, B * (D // 16), zero_buf, 0)
    r0 = s * RPT

    def zero_acc(i, carry):
        pltpu.sync_copy(rows[0], acc_sh.at[pl.ds(r0 + i * B, B)])
        return carry

    nfull = RPT // B
    lax.fori_loop(0, nfull, zero_acc, 0)
    tail = RPT - nfull * B
    pltpu.sync_copy(rows[0].at[pl.ds(0, tail)],
                    acc_sh.at[pl.ds(r0 + nfull * B, tail)])
    plsc.subcore_barrier()

    def step(i, carry):
        p = lax.rem(i, 2)

        @pl.when(i + 1 < NSTEP)
        def _prefetch():
            @pl.when(p == 0)
            def _():
                fetch_idx(i + 1, 1)

            @pl.when(p == 1)
            def _():
                fetch_idx(i + 1, 0)

        def run_set(p):
            drain_idx(p)
            gd = [
                pltpu.async_copy(x_hbm.at[sidx[p].at[j]], rows[j], gsem[j])
                for j in range(K)
            ]
            for j in range(K):
                gd[j].wait()
                pltpu.sync_copy(rows[j], acc_sh.at[didx[p].at[j]], add=True)

        @pl.when(p == 0)
        def _():
            run_set(0)

        @pl.when(p == 1)
        def _():
            run_set(1)

        return carry

    lax.fori_loop(0, NSTEP, step, 0)
    plsc.subcore_barrier()

    pltpu.sync_copy(acc_sh.at[pl.ds(r0, RPT)], out_hbm.at[c, pl.ds(r0, RPT)])


def _mlp_body(x_ref, p_ref, w1_ref, b1_ref, w2_ref, b2_ref, o_ref):
    h = x_ref[...] + p_ref[0] + p_ref[1]
    h1 = jnp.dot(h, w1_ref[...], preferred_element_type=jnp.float32)
    h1 = jnp.maximum(h1 + b1_ref[...], 0.0)
    o_ref[...] = (
        jnp.dot(h1, w2_ref[...], preferred_element_type=jnp.float32)
        + b2_ref[...]
    )


_ROWS_BLK = 1000


def _mlp(x, partials, W1, b1, W2, b2):
    grid = (N // _ROWS_BLK,)
    return pl.pallas_call(
        _mlp_body,
        grid=grid,
        in_specs=[
            pl.BlockSpec((_ROWS_BLK, D), lambda i: (i, 0)),
            pl.BlockSpec((NC, _ROWS_BLK, D), lambda i: (0, i, 0)),
            pl.BlockSpec((D, D), lambda i: (0, 0)),
            pl.BlockSpec((1, D), lambda i: (0, 0)),
            pl.BlockSpec((D, D), lambda i: (0, 0)),
            pl.BlockSpec((1, D), lambda i: (0, 0)),
        ],
        out_specs=pl.BlockSpec((_ROWS_BLK, D), lambda i: (i, 0)),
        out_shape=jax.ShapeDtypeStruct((N, D), jnp.float32),
    )(x, partials, W1, b1, W2, b2)


def kernel(x, edge_index, W1, b1, W2, b2):
    src = edge_index[0].reshape(NW, EPW)
    dst = edge_index[1].reshape(NW, EPW)
    src = jnp.concatenate(
        [src, jnp.zeros((NW, PADE), jnp.int32)], axis=1
    ).reshape(NW, NSTEP, K, B)
    pad_dst = jnp.broadcast_to(
        N + (jnp.arange(PADE, dtype=jnp.int32) % (NPAD - N)), (NW, PADE))
    dst = jnp.concatenate([dst, pad_dst], axis=1).reshape(NW, NSTEP, K, B)
    partials = _sc_agg(x, src, dst)
    return _mlp(x, partials, W1, b1.reshape(1, D), W2, b2.reshape(1, D))


# speedup vs baseline: 1.8886x; 1.1122x over previous
import functools

import jax
import jax.numpy as jnp
from jax import lax
from jax.experimental import pallas as pl
from jax.experimental.pallas import tpu as pltpu
from jax.experimental.pallas import tpu_sc as plsc

N, E, D = 10000, 320000, 128
NC, NS = 2, 16
NW = NC * NS
EPW = E // NW
B = 120
NBP = 84
PADE = NBP * B - EPW
K = 3
NSTEP = NBP // K
NPAD = 10240
RPT = NPAD // NS


@functools.partial(
    pl.kernel,
    out_type=jax.ShapeDtypeStruct((NC, NPAD, D), jnp.float32),
    mesh=plsc.VectorSubcoreMesh(core_axis_name="c", subcore_axis_name="s"),
    scratch_types=(
        [pltpu.VMEM((K, B), jnp.int32)] * 2
        + [pltpu.VMEM((K, B), jnp.int32)] * 2
        + [pltpu.VMEM((B, D), jnp.float32)] * K
        + [pltpu.VMEM_SHARED((NPAD, D), jnp.float32)]
        + [pltpu.SemaphoreType.DMA] * (2 * K + 2)
    ),
)
def _sc_agg(x_hbm, src_hbm, dst_hbm, out_hbm, *rest):
    sidx = rest[0:2]
    didx = rest[2:4]
    rows = rest[4:4 + K]
    acc_sh = rest[4 + K]
    gsem = rest[5 + K:5 + 2 * K]
    ssem = rest[5 + 2 * K:5 + 3 * K]
    isem = rest[5 + 3 * K:7 + 3 * K]

    c = lax.axis_index("c")
    s = lax.axis_index("s")
    wid = s * NC + c

    def fetch_idx(i, p):
        pltpu.async_copy(src_hbm.at[wid, i], sidx[p], isem[p])
        pltpu.async_copy(dst_hbm.at[wid, i], didx[p], isem[p])

    def drain_idx(p):
        pltpu.make_async_copy(src_hbm.at[wid, 0], sidx[p], isem[p]).wait()
        pltpu.make_async_copy(dst_hbm.at[wid, 0], didx[p], isem[p]).wait()

    fetch_idx(0, 0)

    zeros16 = jnp.zeros((16,), jnp.float32)

    def zero_buf(j, carry):
        rows[0][j // (D // 16), pl.ds((j % (D // 16)) * 16, 16)] = zeros16
        return carry

    lax.fori_loop(0, B * (D // 16), zero_buf, 0)
    r0 = s * RPT

    def zero_acc(i, carry):
        pltpu.sync_copy(rows[0], acc_sh.at[pl.ds(r0 + i * B, B)])
        return carry

    nfull = RPT // B
    lax.fori_loop(0, nfull, zero_acc, 0)
    tail = RPT - nfull * B
    pltpu.sync_copy(rows[0].at[pl.ds(0, tail)],
                    acc_sh.at[pl.ds(r0 + nfull * B, tail)])
    plsc.subcore_barrier()

    def step(i, carry):
        p = lax.rem(i, 2)

        @pl.when(i + 1 < NSTEP)
        def _prefetch():
            @pl.when(p == 0)
            def _():
                fetch_idx(i + 1, 1)

            @pl.when(p == 1)
            def _():
                fetch_idx(i + 1, 0)

        def run_set(p):
            drain_idx(p)
            gd = []
            for j in range(K):
                @pl.when(i > 0)
                def _(j=j, p=p):
                    pltpu.make_async_copy(
                        rows[j], acc_sh.at[didx[p].at[j]], ssem[j]).wait()

                gd.append(pltpu.async_copy(
                    x_hbm.at[sidx[p].at[j]], rows[j], gsem[j]))
            for j in range(K):
                gd[j].wait()
                pltpu.async_copy(
                    rows[j], acc_sh.at[didx[p].at[j]], ssem[j], add=True)

        @pl.when(p == 0)
        def _():
            run_set(0)

        @pl.when(p == 1)
        def _():
            run_set(1)

        return carry

    lax.fori_loop(0, NSTEP, step, 0)
    for j in range(K):
        pltpu.make_async_copy(
            rows[j], acc_sh.at[didx[0].at[j]], ssem[j]).wait()
    plsc.subcore_barrier()

    pltpu.sync_copy(acc_sh.at[pl.ds(r0, RPT)], out_hbm.at[c, pl.ds(r0, RPT)])


def _mlp_body(x_ref, p_ref, w1_ref, b1_ref, w2_ref, b2_ref, o_ref):
    h = x_ref[...] + p_ref[0] + p_ref[1]
    h1 = jnp.dot(h, w1_ref[...], preferred_element_type=jnp.float32)
    h1 = jnp.maximum(h1 + b1_ref[...], 0.0)
    o_ref[...] = (
        jnp.dot(h1, w2_ref[...], preferred_element_type=jnp.float32)
        + b2_ref[...]
    )


_ROWS_BLK = 1000


def _mlp(x, partials, W1, b1, W2, b2):
    grid = (N // _ROWS_BLK,)
    return pl.pallas_call(
        _mlp_body,
        grid=grid,
        in_specs=[
            pl.BlockSpec((_ROWS_BLK, D), lambda i: (i, 0)),
            pl.BlockSpec((NC, _ROWS_BLK, D), lambda i: (0, i, 0)),
            pl.BlockSpec((D, D), lambda i: (0, 0)),
            pl.BlockSpec((1, D), lambda i: (0, 0)),
            pl.BlockSpec((D, D), lambda i: (0, 0)),
            pl.BlockSpec((1, D), lambda i: (0, 0)),
        ],
        out_specs=pl.BlockSpec((_ROWS_BLK, D), lambda i: (i, 0)),
        out_shape=jax.ShapeDtypeStruct((N, D), jnp.float32),
    )(x, partials, W1, b1, W2, b2)


def kernel(x, edge_index, W1, b1, W2, b2):
    src = edge_index[0].reshape(NW, EPW)
    dst = edge_index[1].reshape(NW, EPW)
    src = jnp.concatenate(
        [src, jnp.zeros((NW, PADE), jnp.int32)], axis=1
    ).reshape(NW, NSTEP, K, B)
    pad_dst = jnp.broadcast_to(
        N + (jnp.arange(PADE, dtype=jnp.int32) % (NPAD - N)), (NW, PADE))
    dst = jnp.concatenate([dst, pad_dst], axis=1).reshape(NW, NSTEP, K, B)
    partials = _sc_agg(x, src, dst)
    return _mlp(x, partials, W1, b1.reshape(1, D), W2, b2.reshape(1, D))
